# baseline (speedup 1.0000x reference)
import functools

import jax
import jax.numpy as jnp
from jax import lax
from jax.experimental import pallas as pl
from jax.experimental.pallas import tpu as pltpu
from jax.experimental.pallas import tpu_sc as plsc

_NCORE = 2
_NSUB = 16
_NW = _NCORE * _NSUB
_C = 128
_NB = 4
_KCA = 80
_KCB = 80


def _sc_mesh():
    return plsc.VectorSubcoreMesh(core_axis_name="c", subcore_axis_name="s")


def _sc_degree_rowmap(row2d, col2d, n_nodes, np_nodes):
    nrows = row2d.shape[0]
    kcd = nrows // _NW
    rt = np_nodes // _NSUB

    @functools.partial(
        pl.kernel,
        out_type=(
            jax.ShapeDtypeStruct((nrows, _C), jnp.int32),
            jax.ShapeDtypeStruct((_NCORE, np_nodes), jnp.float32),
        ),
        mesh=_sc_mesh(),
        scratch_types=[
            pltpu.VMEM((kcd, _C), jnp.int32),
            pltpu.VMEM((kcd, _C), jnp.int32),
            pltpu.VMEM((kcd, _C), jnp.int32),
            pltpu.VMEM((kcd, _C), jnp.float32),
            pltpu.VMEM((rt,), jnp.float32),
            pltpu.VMEM_SHARED((np_nodes,), jnp.float32),
            pltpu.SemaphoreType.DMA,
        ],
    )
    def k(row_hbm, col_hbm, rowmap_hbm, degp_hbm, rbuf, cbuf, rmbuf, wbuf,
          stage, degacc, sem):
        core = lax.axis_index("c")
        sub = lax.axis_index("s")
        wid = core * _NSUB + sub

        zeros16 = jnp.zeros((16,), jnp.float32)
        ones16 = jnp.ones((16,), jnp.float32)
        dummy16 = jnp.full((16,), n_nodes, jnp.int32)

        def zfill(i, _):
            stage[pl.ds(i * 16, 16)] = zeros16
            return 0

        lax.fori_loop(0, rt // 16, zfill, 0)
        pltpu.sync_copy(stage, degacc.at[pl.ds(sub * rt, rt)])

        pltpu.sync_copy(row_hbm.at[pl.ds(wid * kcd, kcd)], rbuf)
        pltpu.sync_copy(col_hbm.at[pl.ds(wid * kcd, kcd)], cbuf)

        def comp(t, _):
            for j in range(_C // 16):
                sl = pl.ds(j * 16, 16)
                r = rbuf[t, sl]
                m = r == cbuf[t, sl]
                rmbuf[t, sl] = jnp.where(m, dummy16, r)
                wbuf[t, sl] = jnp.where(m, zeros16, ones16)
            return 0

        lax.fori_loop(0, kcd, comp, 0)
        pltpu.sync_copy(rmbuf, rowmap_hbm.at[pl.ds(wid * kcd, kcd)])
        plsc.subcore_barrier()

        def fire(t, _):
            pltpu.async_copy(wbuf.at[t], degacc.at[rbuf.at[t]], sem, add=True)
            return 0

        lax.fori_loop(0, kcd, fire, 0)

        def drain(t, _):
            pltpu.make_async_copy(wbuf.at[t], degacc.at[rbuf.at[t]], sem).wait()
            return 0

        lax.fori_loop(0, kcd, drain, 0)
        plsc.subcore_barrier()
        pltpu.sync_copy(degacc.at[pl.ds(sub * rt, rt)], stage)
        pltpu.sync_copy(stage, degp_hbm.at[core, pl.ds(sub * rt, rt)])

    return k(row2d, col2d)


def _sc_propagate(z_pad, rowmapf, colf, np_nodes):
    d = z_pad.shape[1]
    rt = np_nodes // _NSUB

    @functools.partial(
        pl.kernel,
        out_type=jax.ShapeDtypeStruct((_NCORE, np_nodes, d), jnp.float32),
        mesh=_sc_mesh(),
        scratch_types=[
            [pltpu.VMEM((_C, d), jnp.float32)] * 2,
            pltpu.VMEM_SHARED((np_nodes, d), jnp.float32),
            [pltpu.SemaphoreType.DMA] * 2,
            [pltpu.SemaphoreType.DMA] * 2,
            [pltpu.SemaphoreType.DMA] * _NB,
            [pltpu.VMEM((_C,), jnp.int32)] * _NB,
            [pltpu.VMEM((_C,), jnp.int32)] * _NB,
        ],
    )
    def k(z_hbm, rmapf_hbm, colf_hbm, out_hbm, rows, acc,
          gsems, ssems, isems, rc, cc):
        core = lax.axis_index("c")
        sub = lax.axis_index("s")
        kc = _KCA
        chunk0 = (core * _NSUB + sub) * _KCA

        zeros16 = jnp.zeros((16,), jnp.float32)

        def idx_copy(t, s, start):
            base = (chunk0 + t) * _C
            for src, dst in ((rmapf_hbm, rc[s]), (colf_hbm, cc[s])):
                cp = pltpu.make_async_copy(src.at[pl.ds(base, _C)], dst,
                                           isems[s])
                if start:
                    cp.start()
                else:
                    cp.wait()

        for s in range(_NB):
            idx_copy(s, s, True)

        def zfill(i, _):
            for j in range(d // 16):
                rows[0][i, pl.ds(j * 16, 16)] = zeros16
            return 0

        lax.fori_loop(0, _C, zfill, 0)
        for i in range(rt // _C):
            pltpu.async_copy(rows[0],
                             acc.at[pl.ds(sub * rt + i * _C, _C)], gsems[1])
        for i in range(rt // _C):
            pltpu.make_async_copy(rows[0],
                                  acc.at[pl.ds(sub * rt, _C)], gsems[1]).wait()

        plsc.subcore_barrier()
        idx_copy(0, 0, False)
        pltpu.async_copy(z_hbm.at[rc[0]], rows[0], gsems[0])

        def scat(ph, start):
            if start:
                pltpu.async_copy(rows[ph % 2], acc.at[cc[ph % _NB]],
                                 ssems[ph % 2], add=True)
            else:
                pltpu.make_async_copy(rows[ph % 2], acc.at[cc[ph % _NB]],
                                      ssems[ph % 2]).wait()

        def group(g, _):
            for bb in range(_NB):
                t = g * _NB + bb
                b2 = bb % 2
                pltpu.make_async_copy(z_hbm.at[rc[bb]], rows[b2],
                                      gsems[b2]).wait()
                tn = t + 1
                sn = (bb + 1) % _NB
                bn2 = (bb + 1) % 2
                sp = (bb - 1) % _NB

                @pl.when(t >= 1)
                def _():
                    scat(bb - 1, False)

                    @pl.when(t + 3 < kc)
                    def _():
                        idx_copy(t + 3, sp, True)

                @pl.when(tn < kc)
                def _():
                    idx_copy(tn, sn, False)
                    pltpu.async_copy(z_hbm.at[rc[sn]], rows[bn2], gsems[bn2])

                scat(bb, True)
            return 0

        lax.fori_loop(0, kc // _NB, group, 0)
        scat((kc - 1) % _NB, False)
        plsc.subcore_barrier()
        nsl = rt // _C

        def rd(i, b):
            r0 = sub * rt + i * _C
            return pltpu.make_async_copy(acc.at[pl.ds(r0, _C)], rows[b],
                                         gsems[b])

        def wr(i, b):
            r0 = sub * rt + i * _C
            return pltpu.make_async_copy(rows[b], out_hbm.at[core,
                                                             pl.ds(r0, _C)],
                                         gsems[b])

        rd(0, 0).start()
        for i in range(nsl):
            b = i % 2
            rd(i, b).wait()
            if i + 1 < nsl:
                rd(i + 1, (i + 1) % 2).start()
            wr(i, b).start()
            wr(i, b).wait()

    return k(z_pad, rowmapf, colf)


def _tc_prescale(deg4, x_pad):
    npn, d = x_pad.shape
    rb = 1024
    nb = npn // rb
    sb = rb // 128

    def body(degr, xr, dinvbr, z1r):
        rid = lax.broadcasted_iota(jnp.int32, (128, 128), 0)
        cid = lax.broadcasted_iota(jnp.int32, (128, 128), 1)
        ones = jnp.ones((128, 128), jnp.float32)
        for j in range(sb):
            dv = degr[0, j] + degr[1, j]
            pos = dv > 0.0
            dinv = jnp.where(pos, lax.rsqrt(jnp.where(pos, dv, 1.0)), 0.0)
            diag = jnp.where(rid == cid,
                             jnp.broadcast_to(dinv, (128, 128)), 0.0)
            dm = jnp.dot(diag, ones, preferred_element_type=jnp.float32)
            sl = pl.ds(j * 128, 128)
            dinvbr[sl, :] = dm
            z1r[sl, :] = dm * xr[sl, :]

    return pl.pallas_call(
        body,
        grid=(nb,),
        in_specs=[
            pl.BlockSpec((2, sb, 1, 128), lambda g: (0, g, 0, 0)),
            pl.BlockSpec((rb, d), lambda g: (g, 0)),
        ],
        out_specs=[
            pl.BlockSpec((rb, 128), lambda g: (g, 0)),
            pl.BlockSpec((rb, d), lambda g: (g, 0)),
        ],
        out_shape=(
            jax.ShapeDtypeStruct((npn, 128), jnp.float32),
            jax.ShapeDtypeStruct((npn, d), jnp.float32),
        ),
    )(deg4, x_pad)


def _tc_mid(a1, dinvb):
    _, npn, d = a1.shape
    rb = 1024
    nb = npn // rb

    def body(ar, dr, u1r, z2r):
        dm = dr[...]
        u1 = -(dm * (ar[0] + ar[1]))
        u1r[...] = u1
        z2r[...] = dm * u1

    return pl.pallas_call(
        body,
        grid=(nb,),
        in_specs=[
            pl.BlockSpec((2, rb, d), lambda g: (0, g, 0)),
            pl.BlockSpec((rb, 128), lambda g: (g, 0)),
        ],
        out_specs=[
            pl.BlockSpec((rb, d), lambda g: (g, 0)),
            pl.BlockSpec((rb, d), lambda g: (g, 0)),
        ],
        out_shape=(
            jax.ShapeDtypeStruct((npn, d), jnp.float32),
            jax.ShapeDtypeStruct((npn, d), jnp.float32),
        ),
    )(a1, dinvb)


def _tc_final(x, u1, a2, dinvb, wa, wb, wc, params):
    n, d = x.shape
    rb = 1000
    nb = n // rb

    def body(xr, u1r, ar, dr, war, wbr, wcr, pr, outr):
        u2 = dr[...] * (ar[0] + ar[1])
        acc = jnp.dot(xr[...], war[...], preferred_element_type=jnp.float32)
        acc += jnp.dot(u1r[...], wbr[...], preferred_element_type=jnp.float32)
        acc += jnp.dot(u2, wcr[...], preferred_element_type=jnp.float32)
        acc += pr[0:1, :]
        mu = jnp.mean(acc, axis=-1, keepdims=True)
        var = jnp.mean((acc - mu) ** 2, axis=-1, keepdims=True)
        outr[...] = (acc - mu) / jnp.sqrt(var + 1e-5) * pr[1:2, :] + pr[2:3, :]

    return pl.pallas_call(
        body,
        grid=(nb,),
        in_specs=[
            pl.BlockSpec((rb, d), lambda g: (g, 0)),
            pl.BlockSpec((rb, d), lambda g: (g, 0)),
            pl.BlockSpec((2, rb, d), lambda g: (0, g, 0)),
            pl.BlockSpec((rb, 128), lambda g: (g, 0)),
            pl.BlockSpec((d, d), lambda g: (0, 0)),
            pl.BlockSpec((d, d), lambda g: (0, 0)),
            pl.BlockSpec((d, d), lambda g: (0, 0)),
            pl.BlockSpec((8, d), lambda g: (0, 0)),
        ],
        out_specs=pl.BlockSpec((rb, d), lambda g: (g, 0)),
        out_shape=jax.ShapeDtypeStruct((n, d), jnp.float32),
    )(x, u1, a2, dinvb, wa, wb, wc, params)


def kernel(features, edge_index, W, b, gamma, beta):
    n, d = features.shape
    e = edge_index.shape[1]
    npn = -(-n // (_NSUB * _C)) * (_NSUB * _C)
    nchunks = _NSUB * (_KCA + _KCB)
    epad = nchunks * _C

    row = edge_index[0].astype(jnp.int32)
    col = edge_index[1].astype(jnp.int32)
    ar = jnp.arange(epad - e, dtype=jnp.int32)
    pad_row = n + ar % (npn - n)
    pad_col = ar % npn
    row2d = jnp.concatenate([row, pad_row]).reshape(nchunks, _C)
    col2d = jnp.concatenate([col, pad_col]).reshape(nchunks, _C)
    x_pad = jnp.pad(features, ((0, npn - n), (0, 0)))

    rowmap2d, deg_parts = _sc_degree_rowmap(row2d, col2d, n, npn)
    deg4 = deg_parts.reshape(_NCORE, npn // 128, 1, 128)
    dinvb, z1 = _tc_prescale(deg4, x_pad)
    a1 = _sc_propagate(z1, rowmap2d.reshape(-1), col2d.reshape(-1), npn)
    u1, z2 = _tc_mid(a1, dinvb)
    a2 = _sc_propagate(z2, rowmap2d.reshape(-1), col2d.reshape(-1), npn)

    wa = W[0] - W[2]
    wb = W[1]
    wc = -2.0 * W[2]
    params = jnp.zeros((8, d), jnp.float32).at[0].set(b).at[1].set(gamma).at[2].set(beta)
    return _tc_final(features, u1, a2, dinvb, wa, wb, wc, params)
